# baseline (speedup 1.0000x reference)
import functools

import jax
import jax.numpy as jnp
from jax import lax
from jax.experimental import pallas as pl
from jax.experimental.pallas import tpu as pltpu
from jax.experimental.pallas import tpu_sc as plsc

N = 10000
E = 320000
F = 128

NC = 2
NS = 16
NW = NC * NS
ET = E // NW
CH = 80
ONES = 80
NCHUNK = ET // CH
RT = 632
RTL = N - 15 * RT
CT = 2000


def _sc_agg_body(with_cnt, feat, src, dst, agg_out, cnt_out,
                 agg_sh, cnt_sh, sidx_all, didx0, didx1,
                 rows0, rows1, ones, zv,
                 g0, g1, s0, s1, c0, c1, d0, d1):
    c = lax.axis_index("c")
    s = lax.axis_index("s")
    ebase = (c * NS + s) * ET
    didx = (didx0, didx1)
    rows = (rows0, rows1)
    gsem = (g0, g1)
    ssem = (s0, s1)
    csem = (c0, c1)
    dsem = (d0, d1)

    def _zrow(i, carry):
        for j in range(F // 16):
            rows0[i, pl.ds(j * 16, 16)] = jnp.zeros((16,), jnp.float32)
        return carry

    lax.fori_loop(0, CH, _zrow, 0)
    ZR = N // NS
    for k in range(ZR // CH):
        pltpu.sync_copy(rows0, agg_sh.at[pl.ds(s * ZR + k * CH, CH)])
    if ZR % CH:
        pltpu.sync_copy(rows0.at[pl.ds(0, ZR % CH)],
                        agg_sh.at[pl.ds(s * ZR + (ZR // CH) * CH, ZR % CH)])
    if with_cnt:
        def _zcnt(i, carry):
            zv[pl.ds(i * 16, 16)] = jnp.zeros((16,), jnp.float32)
            return carry

        lax.fori_loop(0, CT // 16, _zcnt, 0)

        @pl.when(s < N // CT)
        def _():
            pltpu.sync_copy(zv, cnt_sh.at[pl.ds(s * CT, CT)])
        for i in range(ONES // 16):
            ones[pl.ds(i * 16, 16)] = jnp.ones((16,), jnp.float32)
    plsc.subcore_barrier()

    def issue_didx(i, b):
        pltpu.async_copy(dst.at[pl.ds(ebase + i * CH, CH)], didx[b], dsem[b])

    def wait_didx(b):
        pltpu.make_async_copy(dst.at[pl.ds(0, CH)], didx[b], dsem[b]).wait()

    def issue_gather(i, b):
        pltpu.async_copy(feat.at[sidx_all.at[pl.ds(i * CH, CH)]],
                         rows[b], gsem[b])

    def wait_gather(b):
        pltpu.make_async_copy(feat.at[sidx_all.at[pl.ds(0, CH)]],
                              rows[b], gsem[b]).wait()

    def issue_scatter(b):
        pltpu.async_copy(rows[b], agg_sh.at[didx[b]], ssem[b], add=True)

    def wait_scatter(b):
        pltpu.make_async_copy(rows[b], agg_sh.at[didx[b]], ssem[b]).wait()

    def issue_cnt(b):
        if with_cnt:
            pltpu.async_copy(ones, cnt_sh.at[didx[b]], csem[b], add=True)

    def wait_cnt(b):
        if with_cnt:
            pltpu.make_async_copy(ones, cnt_sh.at[didx[b]], csem[b]).wait()

    pltpu.sync_copy(src.at[pl.ds(ebase, ET)], sidx_all)
    issue_didx(0, 0)
    issue_gather(0, 0)
    wait_gather(0)
    wait_didx(0)
    issue_scatter(0)
    issue_cnt(0)
    issue_didx(1, 1)
    issue_gather(1, 1)

    def step(i, b, do_next):
        nb = 1 - b
        wait_gather(b)
        wait_didx(b)
        issue_scatter(b)
        issue_cnt(b)
        wait_scatter(nb)
        wait_cnt(nb)
        if do_next:
            issue_didx(i + 1, nb)
            issue_gather(i + 1, nb)

    def pair(p, carry):
        i0 = 2 * p + 1
        step(i0, 1, True)
        step(i0 + 1, 0, True)
        return carry

    lax.fori_loop(0, (NCHUNK - 3) // 2, pair, 0)
    step(NCHUNK - 2, (NCHUNK - 2) % 2, True)
    step(NCHUNK - 1, (NCHUNK - 1) % 2, False)
    wait_scatter((NCHUNK - 1) % 2)
    wait_cnt((NCHUNK - 1) % 2)
    plsc.subcore_barrier()

    @pl.when(s < NS - 1)
    def _():
        pltpu.sync_copy(agg_sh.at[pl.ds(s * RT, RT)],
                        agg_out.at[pl.ds(c * N + s * RT, RT)])

    @pl.when(s == NS - 1)
    def _():
        pltpu.sync_copy(agg_sh.at[pl.ds(s * RT, RTL)],
                        agg_out.at[pl.ds(c * N + s * RT, RTL)])
    if with_cnt:
        @pl.when(s < N // CT)
        def _():
            pltpu.sync_copy(cnt_sh.at[pl.ds(s * CT, CT)], zv)
            pltpu.sync_copy(zv, cnt_out.at[pl.ds(c * N + s * CT, CT)])


def _make_sc_agg(with_cnt):
    mesh = plsc.VectorSubcoreMesh(core_axis_name="c", subcore_axis_name="s",
                                  num_cores=NC, num_subcores=NS)
    return pl.kernel(
        functools.partial(_sc_agg_body, with_cnt),
        out_type=(
            jax.ShapeDtypeStruct((NC * N, F), jnp.float32),
            jax.ShapeDtypeStruct((NC * N,), jnp.float32),
        ),
        mesh=mesh,
        scratch_types=[
            pltpu.VMEM_SHARED((N, F), jnp.float32),
            pltpu.VMEM_SHARED((N,), jnp.float32),
            pltpu.VMEM((ET,), jnp.int32),
            pltpu.VMEM((CH,), jnp.int32),
            pltpu.VMEM((CH,), jnp.int32),
            pltpu.VMEM((CH, F), jnp.float32),
            pltpu.VMEM((CH, F), jnp.float32),
            pltpu.VMEM((ONES,), jnp.float32),
            pltpu.VMEM((CT,), jnp.float32),
            pltpu.SemaphoreType.DMA,
            pltpu.SemaphoreType.DMA,
            pltpu.SemaphoreType.DMA,
            pltpu.SemaphoreType.DMA,
            pltpu.SemaphoreType.DMA,
            pltpu.SemaphoreType.DMA,
            pltpu.SemaphoreType.DMA,
            pltpu.SemaphoreType.DMA,
        ],
        name="sage_sc_agg" + ("_cnt" if with_cnt else ""),
    )


_sc_agg_cnt = _make_sc_agg(True)
_sc_agg = _make_sc_agg(False)

BR = 2000


def _tc1_body(agg_ref, cnt_ref, x_ref, wl_ref, bl_ref, wr_ref,
              gm_ref, bt_ref, rm_ref, rv_ref, o_ref):
    agg = agg_ref[0] + agg_ref[1]
    cnt = cnt_ref[0] + cnt_ref[1]
    rinv = 1.0 / jnp.maximum(cnt, 1.0)
    z = (jnp.dot(agg * rinv, wl_ref[...], precision=lax.Precision.HIGHEST,
                 preferred_element_type=jnp.float32)
         + jnp.dot(x_ref[...], wr_ref[...], precision=lax.Precision.HIGHEST,
                   preferred_element_type=jnp.float32)
         + bl_ref[...])
    sc = gm_ref[...] * lax.rsqrt(rv_ref[...] + 1e-5)
    sh = bt_ref[...] - rm_ref[...] * sc
    o_ref[...] = jnp.maximum(z * sc + sh, 0.0)


def _tc2_body(agg_ref, cnt_ref, h_ref, wl_ref, bl_ref, wr_ref, o_ref):
    agg = agg_ref[0] + agg_ref[1]
    cnt = cnt_ref[0] + cnt_ref[1]
    rinv = 1.0 / jnp.maximum(cnt, 1.0)
    o_ref[...] = (jnp.dot(agg * rinv, wl_ref[...],
                          precision=lax.Precision.HIGHEST,
                          preferred_element_type=jnp.float32)
                  + jnp.dot(h_ref[...], wr_ref[...],
                            precision=lax.Precision.HIGHEST,
                            preferred_element_type=jnp.float32)
                  + bl_ref[...])


_row_spec = pl.BlockSpec((BR, F), lambda i: (i, 0))
_agg_spec = pl.BlockSpec((NC, BR, F), lambda i: (0, i, 0))
_cnt_spec = pl.BlockSpec((NC, BR, 1), lambda i: (0, i, 0))
_vec_spec = pl.BlockSpec((1, F), lambda i: (0, 0))


def _tc1(agg, cnt, x, wl, bl, wr, gm, bt, rm, rv):
    return pl.pallas_call(
        _tc1_body,
        grid=(N // BR,),
        in_specs=[_agg_spec, _cnt_spec, _row_spec] + [_vec_spec] * 0 +
                 [pl.BlockSpec((F, F), lambda i: (0, 0)), _vec_spec,
                  pl.BlockSpec((F, F), lambda i: (0, 0)),
                  _vec_spec, _vec_spec, _vec_spec, _vec_spec],
        out_specs=_row_spec,
        out_shape=jax.ShapeDtypeStruct((N, F), jnp.float32),
    )(agg, cnt, x, wl, bl, wr, gm, bt, rm, rv)


def _tc2(agg, cnt, h, wl, bl, wr):
    return pl.pallas_call(
        _tc2_body,
        grid=(N // BR,),
        in_specs=[_agg_spec, _cnt_spec, _row_spec,
                  pl.BlockSpec((F, F), lambda i: (0, 0)), _vec_spec,
                  pl.BlockSpec((F, F), lambda i: (0, 0))],
        out_specs=_row_spec,
        out_shape=jax.ShapeDtypeStruct((N, F), jnp.float32),
    )(agg, cnt, h, wl, bl, wr)


def kernel(x, ei, W1l, b1l, W1r, gamma, beta, rm, rv, W2l, b2l, W2r):
    src = ei[0]
    dst = ei[1]

    agg1, cnt = _sc_agg_cnt(x, src, dst)
    agg1 = agg1.reshape(NC, N, F)
    cnt3 = cnt.reshape(NC, N, 1)
    h = _tc1(agg1, cnt3, x, W1l, b1l.reshape(1, F), W1r,
             gamma.reshape(1, F), beta.reshape(1, F),
             rm.reshape(1, F), rv.reshape(1, F))

    agg2, _ = _sc_agg(h, src, dst)
    agg2 = agg2.reshape(NC, N, F)
    out = _tc2(agg2, cnt3, h, W2l, b2l.reshape(1, F), W2r)
    return out


# speedup vs baseline: 1.0065x; 1.0065x over previous
import functools

import jax
import jax.numpy as jnp
from jax import lax
from jax.experimental import pallas as pl
from jax.experimental.pallas import tpu as pltpu
from jax.experimental.pallas import tpu_sc as plsc

N = 10000
E = 320000
F = 128

NC = 2
NS = 16
NW = NC * NS
ET = E // NW
CH = 80
ONES = 80
NCHUNK = ET // CH
RT = 632
RTL = N - 15 * RT
CT = 2000


def _sc_agg_body(with_cnt, feat, src, dst, agg_out, cnt_out,
                 agg_sh, cnt_sh, sidx_all, didx0, didx1,
                 rows0, rows1, ones, zv,
                 g0, g1, s0, s1, c0, c1, d0, d1):
    c = lax.axis_index("c")
    s = lax.axis_index("s")
    ebase = (c * NS + s) * ET
    didx = (didx0, didx1)
    rows = (rows0, rows1)
    gsem = (g0, g1)
    ssem = (s0, s1)
    csem = (c0, c1)
    dsem = (d0, d1)

    def _zrow(i, carry):
        for j in range(F // 16):
            rows0[i, pl.ds(j * 16, 16)] = jnp.zeros((16,), jnp.float32)
        return carry

    lax.fori_loop(0, CH, _zrow, 0)
    ZR = N // NS
    for k in range(ZR // CH):
        pltpu.sync_copy(rows0, agg_sh.at[pl.ds(s * ZR + k * CH, CH)])
    if ZR % CH:
        pltpu.sync_copy(rows0.at[pl.ds(0, ZR % CH)],
                        agg_sh.at[pl.ds(s * ZR + (ZR // CH) * CH, ZR % CH)])
    if with_cnt:
        def _zcnt(i, carry):
            zv[pl.ds(i * 16, 16)] = jnp.zeros((16,), jnp.float32)
            return carry

        lax.fori_loop(0, CT // 16, _zcnt, 0)

        @pl.when(s < N // CT)
        def _():
            pltpu.sync_copy(zv, cnt_sh.at[pl.ds(s * CT, CT)])
        for i in range(ONES // 16):
            ones[pl.ds(i * 16, 16)] = jnp.ones((16,), jnp.float32)
    plsc.subcore_barrier()

    def issue_didx(i, b):
        pltpu.async_copy(dst.at[pl.ds(ebase + i * CH, CH)], didx[b], dsem[b])

    def wait_didx(b):
        pltpu.make_async_copy(dst.at[pl.ds(0, CH)], didx[b], dsem[b]).wait()

    def issue_gather(i, b):
        pltpu.async_copy(feat.at[sidx_all.at[pl.ds(i * CH, CH)]],
                         rows[b], gsem[b])

    def wait_gather(b):
        pltpu.make_async_copy(feat.at[sidx_all.at[pl.ds(0, CH)]],
                              rows[b], gsem[b]).wait()

    def issue_scatter(b):
        pltpu.async_copy(rows[b], agg_sh.at[didx[b]], ssem[b], add=True)

    def wait_scatter(b):
        pltpu.make_async_copy(rows[b], agg_sh.at[didx[b]], ssem[b]).wait()

    def issue_cnt(b):
        if with_cnt:
            pltpu.async_copy(ones, cnt_sh.at[didx[b]], csem[b], add=True)

    def wait_cnt(b):
        if with_cnt:
            pltpu.make_async_copy(ones, cnt_sh.at[didx[b]], csem[b]).wait()

    pltpu.sync_copy(src.at[pl.ds(ebase, ET)], sidx_all)
    issue_didx(0, 0)
    issue_gather(0, 0)
    wait_gather(0)
    wait_didx(0)
    issue_scatter(0)
    issue_cnt(0)
    issue_didx(1, 1)
    issue_gather(1, 1)

    def step(i, b, do_next):
        nb = 1 - b
        wait_gather(b)
        wait_didx(b)
        issue_scatter(b)
        issue_cnt(b)
        wait_scatter(nb)
        wait_cnt(nb)
        if do_next:
            issue_didx(i + 1, nb)
            issue_gather(i + 1, nb)

    def pair(p, carry):
        i0 = 2 * p + 1
        step(i0, 1, True)
        step(i0 + 1, 0, True)
        return carry

    lax.fori_loop(0, (NCHUNK - 3) // 2, pair, 0)
    step(NCHUNK - 2, (NCHUNK - 2) % 2, True)
    step(NCHUNK - 1, (NCHUNK - 1) % 2, False)
    wait_scatter((NCHUNK - 1) % 2)
    wait_cnt((NCHUNK - 1) % 2)
    plsc.subcore_barrier()

    @pl.when(s < NS - 1)
    def _():
        pltpu.sync_copy(agg_sh.at[pl.ds(s * RT, RT)],
                        agg_out.at[pl.ds(c * N + s * RT, RT)])

    @pl.when(s == NS - 1)
    def _():
        pltpu.sync_copy(agg_sh.at[pl.ds(s * RT, RTL)],
                        agg_out.at[pl.ds(c * N + s * RT, RTL)])
    if with_cnt:
        @pl.when(s < N // CT)
        def _():
            pltpu.sync_copy(cnt_sh.at[pl.ds(s * CT, CT)], zv)
            pltpu.sync_copy(zv, cnt_out.at[pl.ds(c * N + s * CT, CT)])


def _make_sc_agg(with_cnt):
    mesh = plsc.VectorSubcoreMesh(core_axis_name="c", subcore_axis_name="s",
                                  num_cores=NC, num_subcores=NS)
    return pl.kernel(
        functools.partial(_sc_agg_body, with_cnt),
        out_type=(
            jax.ShapeDtypeStruct((NC * N, F), jnp.float32),
            jax.ShapeDtypeStruct((NC * N,), jnp.float32),
        ),
        mesh=mesh,
        scratch_types=[
            pltpu.VMEM_SHARED((N, F), jnp.float32),
            pltpu.VMEM_SHARED((N,), jnp.float32),
            pltpu.VMEM((ET,), jnp.int32),
            pltpu.VMEM((CH,), jnp.int32),
            pltpu.VMEM((CH,), jnp.int32),
            pltpu.VMEM((CH, F), jnp.float32),
            pltpu.VMEM((CH, F), jnp.float32),
            pltpu.VMEM((ONES,), jnp.float32),
            pltpu.VMEM((CT,), jnp.float32),
            pltpu.SemaphoreType.DMA,
            pltpu.SemaphoreType.DMA,
            pltpu.SemaphoreType.DMA,
            pltpu.SemaphoreType.DMA,
            pltpu.SemaphoreType.DMA,
            pltpu.SemaphoreType.DMA,
            pltpu.SemaphoreType.DMA,
            pltpu.SemaphoreType.DMA,
        ],
        name="sage_sc_agg" + ("_cnt" if with_cnt else ""),
    )


_sc_agg_cnt = _make_sc_agg(True)
_sc_agg = _make_sc_agg(False)

BR = 2000


def _tc_self_body(x_ref, w_ref, o_ref):
    o_ref[...] = jnp.dot(x_ref[...], w_ref[...],
                         precision=lax.Precision.HIGHEST,
                         preferred_element_type=jnp.float32)


def _tc1_body(agg_ref, cnt_ref, self_ref, wl_ref, bl_ref,
              gm_ref, bt_ref, rm_ref, rv_ref, o_ref):
    agg = agg_ref[0] + agg_ref[1]
    cnt = cnt_ref[0] + cnt_ref[1]
    rinv = 1.0 / jnp.maximum(cnt, 1.0)
    z = (jnp.dot(agg * rinv, wl_ref[...], precision=lax.Precision.HIGHEST,
                 preferred_element_type=jnp.float32)
         + self_ref[...] + bl_ref[...])
    sc = gm_ref[...] * lax.rsqrt(rv_ref[...] + 1e-5)
    sh = bt_ref[...] - rm_ref[...] * sc
    o_ref[...] = jnp.maximum(z * sc + sh, 0.0)


def _tc2_body(agg_ref, cnt_ref, self_ref, wl_ref, bl_ref, o_ref):
    agg = agg_ref[0] + agg_ref[1]
    cnt = cnt_ref[0] + cnt_ref[1]
    rinv = 1.0 / jnp.maximum(cnt, 1.0)
    o_ref[...] = (jnp.dot(agg * rinv, wl_ref[...],
                          precision=lax.Precision.HIGHEST,
                          preferred_element_type=jnp.float32)
                  + self_ref[...] + bl_ref[...])


_row_spec = pl.BlockSpec((BR, F), lambda i: (i, 0))
_agg_spec = pl.BlockSpec((NC, BR, F), lambda i: (0, i, 0))
_cnt_spec = pl.BlockSpec((NC, BR, 1), lambda i: (0, i, 0))
_mat_spec = pl.BlockSpec((F, F), lambda i: (0, 0))
_vec_spec = pl.BlockSpec((1, F), lambda i: (0, 0))


def _tc_self(x, w):
    return pl.pallas_call(
        _tc_self_body,
        grid=(N // BR,),
        in_specs=[_row_spec, _mat_spec],
        out_specs=_row_spec,
        out_shape=jax.ShapeDtypeStruct((N, F), jnp.float32),
    )(x, w)


def _tc1(agg, cnt, self1, wl, bl, gm, bt, rm, rv):
    return pl.pallas_call(
        _tc1_body,
        grid=(N // BR,),
        in_specs=[_agg_spec, _cnt_spec, _row_spec, _mat_spec, _vec_spec,
                  _vec_spec, _vec_spec, _vec_spec, _vec_spec],
        out_specs=_row_spec,
        out_shape=jax.ShapeDtypeStruct((N, F), jnp.float32),
    )(agg, cnt, self1, wl, bl, gm, bt, rm, rv)


def _tc2(agg, cnt, self2, wl, bl):
    return pl.pallas_call(
        _tc2_body,
        grid=(N // BR,),
        in_specs=[_agg_spec, _cnt_spec, _row_spec, _mat_spec, _vec_spec],
        out_specs=_row_spec,
        out_shape=jax.ShapeDtypeStruct((N, F), jnp.float32),
    )(agg, cnt, self2, wl, bl)


def kernel(x, ei, W1l, b1l, W1r, gamma, beta, rm, rv, W2l, b2l, W2r):
    src = ei[0]
    dst = ei[1]

    agg1, cnt = _sc_agg_cnt(x, src, dst)
    self1 = _tc_self(x, W1r)
    agg1 = agg1.reshape(NC, N, F)
    cnt3 = cnt.reshape(NC, N, 1)
    h = _tc1(agg1, cnt3, self1, W1l, b1l.reshape(1, F),
             gamma.reshape(1, F), beta.reshape(1, F),
             rm.reshape(1, F), rv.reshape(1, F))

    agg2, _ = _sc_agg(h, src, dst)
    self2 = _tc_self(h, W2r)
    agg2 = agg2.reshape(NC, N, F)
    out = _tc2(agg2, cnt3, self2, W2l, b2l.reshape(1, F))
    return out


# speedup vs baseline: 1.0473x; 1.0406x over previous
import functools

import jax
import jax.numpy as jnp
from jax import lax
from jax.experimental import pallas as pl
from jax.experimental.pallas import tpu as pltpu
from jax.experimental.pallas import tpu_sc as plsc

N = 10000
E = 320000
F = 128

NC = 2
NS = 16
NW = NC * NS
ET = E // NW
CH = 80
ONES = 80
NCHUNK = ET // CH
RT = 632
RTL = N - 15 * RT
CT = 2000


def _sc_agg_body(with_cnt, feat, eidx, agg_out, cnt_out,
                 agg_sh, cnt_sh, sidx_all, didx0, didx1,
                 rows0, rows1, ones, zv,
                 g0, g1, s0, s1, c0, c1, d0, d1):
    c = lax.axis_index("c")
    s = lax.axis_index("s")
    ebase = (c * NS + s) * ET
    didx = (didx0, didx1)
    rows = (rows0, rows1)
    gsem = (g0, g1)
    ssem = (s0, s1)
    csem = (c0, c1)
    dsem = (d0, d1)

    def _zrow(i, carry):
        for j in range(F // 16):
            rows0[i, pl.ds(j * 16, 16)] = jnp.zeros((16,), jnp.float32)
        return carry

    lax.fori_loop(0, CH, _zrow, 0)
    ZR = N // NS
    for k in range(ZR // CH):
        pltpu.sync_copy(rows0, agg_sh.at[pl.ds(s * ZR + k * CH, CH)])
    if ZR % CH:
        pltpu.sync_copy(rows0.at[pl.ds(0, ZR % CH)],
                        agg_sh.at[pl.ds(s * ZR + (ZR // CH) * CH, ZR % CH)])
    if with_cnt:
        def _zcnt(i, carry):
            zv[pl.ds(i * 16, 16)] = jnp.zeros((16,), jnp.float32)
            return carry

        lax.fori_loop(0, CT // 16, _zcnt, 0)

        @pl.when(s < N // CT)
        def _():
            pltpu.sync_copy(zv, cnt_sh.at[pl.ds(s * CT, CT)])
        for i in range(ONES // 16):
            ones[pl.ds(i * 16, 16)] = jnp.ones((16,), jnp.float32)
    plsc.subcore_barrier()

    def issue_didx(i, b):
        pltpu.async_copy(eidx.at[pl.ds(E + ebase + i * CH, CH)],
                         didx[b], dsem[b])

    def wait_didx(b):
        pltpu.make_async_copy(eidx.at[pl.ds(0, CH)], didx[b], dsem[b]).wait()

    def issue_gather(i, b):
        pltpu.async_copy(feat.at[sidx_all.at[pl.ds(i * CH, CH)]],
                         rows[b], gsem[b])

    def wait_gather(b):
        pltpu.make_async_copy(feat.at[sidx_all.at[pl.ds(0, CH)]],
                              rows[b], gsem[b]).wait()

    def issue_scatter(b):
        pltpu.async_copy(rows[b], agg_sh.at[didx[b]], ssem[b], add=True)

    def wait_scatter(b):
        pltpu.make_async_copy(rows[b], agg_sh.at[didx[b]], ssem[b]).wait()

    def issue_cnt(b):
        if with_cnt:
            pltpu.async_copy(ones, cnt_sh.at[didx[b]], csem[b], add=True)

    def wait_cnt(b):
        if with_cnt:
            pltpu.make_async_copy(ones, cnt_sh.at[didx[b]], csem[b]).wait()

    pltpu.sync_copy(eidx.at[pl.ds(ebase, ET)], sidx_all)
    issue_didx(0, 0)
    issue_gather(0, 0)
    wait_gather(0)
    wait_didx(0)
    issue_scatter(0)
    issue_cnt(0)
    issue_didx(1, 1)
    issue_gather(1, 1)

    def step(i, b, do_next):
        nb = 1 - b
        wait_gather(b)
        wait_didx(b)
        issue_scatter(b)
        issue_cnt(b)
        wait_scatter(nb)
        wait_cnt(nb)
        if do_next:
            issue_didx(i + 1, nb)
            issue_gather(i + 1, nb)

    def pair(p, carry):
        i0 = 2 * p + 1
        step(i0, 1, True)
        step(i0 + 1, 0, True)
        return carry

    lax.fori_loop(0, (NCHUNK - 3) // 2, pair, 0)
    step(NCHUNK - 2, (NCHUNK - 2) % 2, True)
    step(NCHUNK - 1, (NCHUNK - 1) % 2, False)
    wait_scatter((NCHUNK - 1) % 2)
    wait_cnt((NCHUNK - 1) % 2)
    plsc.subcore_barrier()

    @pl.when(s < NS - 1)
    def _():
        pltpu.sync_copy(agg_sh.at[pl.ds(s * RT, RT)],
                        agg_out.at[c, pl.ds(s * RT, RT)])

    @pl.when(s == NS - 1)
    def _():
        pltpu.sync_copy(agg_sh.at[pl.ds(s * RT, RTL)],
                        agg_out.at[c, pl.ds(s * RT, RTL)])
    if with_cnt:
        @pl.when(s < N // CT)
        def _():
            pltpu.sync_copy(cnt_sh.at[pl.ds(s * CT, CT)], zv)
            pltpu.sync_copy(zv, cnt_out.at[pl.ds(c * N + s * CT, CT)])


def _make_sc_agg(with_cnt):
    mesh = plsc.VectorSubcoreMesh(core_axis_name="c", subcore_axis_name="s",
                                  num_cores=NC, num_subcores=NS)
    return pl.kernel(
        functools.partial(_sc_agg_body, with_cnt),
        out_type=(
            jax.ShapeDtypeStruct((NC, N, F), jnp.float32),
            jax.ShapeDtypeStruct((NC * N,), jnp.float32),
        ),
        mesh=mesh,
        scratch_types=[
            pltpu.VMEM_SHARED((N, F), jnp.float32),
            pltpu.VMEM_SHARED((N,), jnp.float32),
            pltpu.VMEM((ET,), jnp.int32),
            pltpu.VMEM((CH,), jnp.int32),
            pltpu.VMEM((CH,), jnp.int32),
            pltpu.VMEM((CH, F), jnp.float32),
            pltpu.VMEM((CH, F), jnp.float32),
            pltpu.VMEM((ONES,), jnp.float32),
            pltpu.VMEM((CT,), jnp.float32),
            pltpu.SemaphoreType.DMA,
            pltpu.SemaphoreType.DMA,
            pltpu.SemaphoreType.DMA,
            pltpu.SemaphoreType.DMA,
            pltpu.SemaphoreType.DMA,
            pltpu.SemaphoreType.DMA,
            pltpu.SemaphoreType.DMA,
            pltpu.SemaphoreType.DMA,
        ],
        name="sage_sc_agg" + ("_cnt" if with_cnt else ""),
    )


_sc_agg_cnt = _make_sc_agg(True)
_sc_agg = _make_sc_agg(False)

BR = 2000


def _tc_self_body(x_ref, w_ref, o_ref):
    o_ref[...] = jnp.dot(x_ref[...], w_ref[...],
                         preferred_element_type=jnp.float32)


def _tc1_body(agg_ref, cnt_ref, self_ref, wl_ref, bl_ref,
              gm_ref, bt_ref, rm_ref, rv_ref, o_ref):
    agg = agg_ref[0] + agg_ref[1]
    cnt = cnt_ref[0] + cnt_ref[1]
    rinv = 1.0 / jnp.maximum(cnt, 1.0)
    z = (jnp.dot(agg * rinv, wl_ref[...],
                 preferred_element_type=jnp.float32)
         + self_ref[...] + bl_ref[...])
    sc = gm_ref[...] * lax.rsqrt(rv_ref[...] + 1e-5)
    sh = bt_ref[...] - rm_ref[...] * sc
    o_ref[...] = jnp.maximum(z * sc + sh, 0.0)


def _tc2_body(agg_ref, cnt_ref, self_ref, wl_ref, bl_ref, o_ref):
    agg = agg_ref[0] + agg_ref[1]
    cnt = cnt_ref[0] + cnt_ref[1]
    rinv = 1.0 / jnp.maximum(cnt, 1.0)
    o_ref[...] = (jnp.dot(agg * rinv, wl_ref[...],
                          preferred_element_type=jnp.float32)
                  + self_ref[...] + bl_ref[...])


_row_spec = pl.BlockSpec((BR, F), lambda i: (i, 0))
_agg_spec = pl.BlockSpec((NC, BR, F), lambda i: (0, i, 0))
_cnt_spec = pl.BlockSpec((NC, BR, 1), lambda i: (0, i, 0))
_mat_spec = pl.BlockSpec((F, F), lambda i: (0, 0))
_vec_spec = pl.BlockSpec((1, F), lambda i: (0, 0))


def _tc_self(x, w):
    return pl.pallas_call(
        _tc_self_body,
        grid=(N // BR,),
        in_specs=[_row_spec, _mat_spec],
        out_specs=_row_spec,
        out_shape=jax.ShapeDtypeStruct((N, F), jnp.float32),
    )(x, w)


def _tc1(agg, cnt, self1, wl, bl, gm, bt, rm, rv):
    return pl.pallas_call(
        _tc1_body,
        grid=(N // BR,),
        in_specs=[_agg_spec, _cnt_spec, _row_spec, _mat_spec, _vec_spec,
                  _vec_spec, _vec_spec, _vec_spec, _vec_spec],
        out_specs=_row_spec,
        out_shape=jax.ShapeDtypeStruct((N, F), jnp.float32),
    )(agg, cnt, self1, wl, bl, gm, bt, rm, rv)


def _tc2(agg, cnt, self2, wl, bl):
    return pl.pallas_call(
        _tc2_body,
        grid=(N // BR,),
        in_specs=[_agg_spec, _cnt_spec, _row_spec, _mat_spec, _vec_spec],
        out_specs=_row_spec,
        out_shape=jax.ShapeDtypeStruct((N, F), jnp.float32),
    )(agg, cnt, self2, wl, bl)


def kernel(x, ei, W1l, b1l, W1r, gamma, beta, rm, rv, W2l, b2l, W2r):
    eidx = ei.reshape(2 * E)

    agg1, cnt = _sc_agg_cnt(x, eidx)
    self1 = _tc_self(x, W1r)
    cnt3 = cnt.reshape(NC, N, 1)
    h = _tc1(agg1, cnt3, self1, W1l, b1l.reshape(1, F),
             gamma.reshape(1, F), beta.reshape(1, F),
             rm.reshape(1, F), rv.reshape(1, F))

    agg2, _ = _sc_agg(h, eidx)
    self2 = _tc_self(h, W2r)
    out = _tc2(agg2, cnt3, self2, W2l, b2l.reshape(1, F))
    return out


# speedup vs baseline: 1.0703x; 1.0219x over previous
import functools

import jax
import jax.numpy as jnp
from jax import lax
from jax.experimental import pallas as pl
from jax.experimental.pallas import tpu as pltpu
from jax.experimental.pallas import tpu_sc as plsc

N = 10000
E = 320000
F = 128

NC = 2
NS = 16
NW = NC * NS
ET = E // NW
CH = 80
ONES = 80
NCHUNK = ET // CH
RT = 632
RTL = N - 15 * RT
CT = 2000


def _sc_agg_body(with_cnt, feat, eidx, agg_out, cnt_out,
                 agg_sh, cnt_sh, sidx_all, didx0, didx1,
                 rows0, rows1, ones, zv,
                 g0, g1, s0, s1, c0, c1, d0, d1):
    c = lax.axis_index("c")
    s = lax.axis_index("s")
    ebase = (c * NS + s) * ET
    didx = (didx0, didx1)
    rows = (rows0, rows1)
    gsem = (g0, g1)
    ssem = (s0, s1)
    csem = (c0, c1)
    dsem = (d0, d1)

    def _zrow(i, carry):
        for j in range(F // 16):
            rows0[i, pl.ds(j * 16, 16)] = jnp.zeros((16,), jnp.float32)
        return carry

    lax.fori_loop(0, CH, _zrow, 0)
    ZR = N // NS
    for k in range(ZR // CH):
        pltpu.sync_copy(rows0, agg_sh.at[pl.ds(s * ZR + k * CH, CH)])
    if ZR % CH:
        pltpu.sync_copy(rows0.at[pl.ds(0, ZR % CH)],
                        agg_sh.at[pl.ds(s * ZR + (ZR // CH) * CH, ZR % CH)])
    if with_cnt:
        def _zcnt(i, carry):
            zv[pl.ds(i * 16, 16)] = jnp.zeros((16,), jnp.float32)
            return carry

        lax.fori_loop(0, CT // 16, _zcnt, 0)

        @pl.when(s < N // CT)
        def _():
            pltpu.sync_copy(zv, cnt_sh.at[pl.ds(s * CT, CT)])
        for i in range(ONES // 16):
            ones[pl.ds(i * 16, 16)] = jnp.ones((16,), jnp.float32)
    plsc.subcore_barrier()

    def issue_didx(i, b):
        pltpu.async_copy(eidx.at[pl.ds(E + ebase + i * CH, CH)],
                         didx[b], dsem[b])

    def wait_didx(b):
        pltpu.make_async_copy(eidx.at[pl.ds(0, CH)], didx[b], dsem[b]).wait()

    def issue_gather(i, b):
        pltpu.async_copy(feat.at[sidx_all.at[pl.ds(i * CH, CH)]],
                         rows[b], gsem[b])

    def wait_gather(b):
        pltpu.make_async_copy(feat.at[sidx_all.at[pl.ds(0, CH)]],
                              rows[b], gsem[b]).wait()

    def issue_scatter(b):
        pltpu.async_copy(rows[b], agg_sh.at[didx[b]], ssem[b], add=True)

    def wait_scatter(b):
        pltpu.make_async_copy(rows[b], agg_sh.at[didx[b]], ssem[b]).wait()

    def issue_cnt(b):
        if with_cnt:
            pltpu.async_copy(ones, cnt_sh.at[didx[b]], csem[b], add=True)

    def wait_cnt(b):
        if with_cnt:
            pltpu.make_async_copy(ones, cnt_sh.at[didx[b]], csem[b]).wait()

    pltpu.sync_copy(eidx.at[pl.ds(ebase, ET)], sidx_all)
    issue_didx(0, 0)
    issue_gather(0, 0)
    wait_gather(0)
    wait_didx(0)
    issue_scatter(0)
    issue_cnt(0)
    issue_didx(1, 1)
    issue_gather(1, 1)

    def step(i, b, do_next):
        nb = 1 - b
        wait_gather(b)
        wait_didx(b)
        issue_scatter(b)
        issue_cnt(b)
        wait_scatter(nb)
        wait_cnt(nb)
        if do_next:
            issue_didx(i + 1, nb)
            issue_gather(i + 1, nb)

    def pair(p, carry):
        i0 = 2 * p + 1
        step(i0, 1, True)
        step(i0 + 1, 0, True)
        return carry

    lax.fori_loop(0, (NCHUNK - 3) // 2, pair, 0)
    step(NCHUNK - 2, (NCHUNK - 2) % 2, True)
    step(NCHUNK - 1, (NCHUNK - 1) % 2, False)
    wait_scatter((NCHUNK - 1) % 2)
    wait_cnt((NCHUNK - 1) % 2)
    plsc.subcore_barrier()

    @pl.when(s < NS - 1)
    def _():
        pltpu.sync_copy(agg_sh.at[pl.ds(s * RT, RT)],
                        agg_out.at[c, pl.ds(s * RT, RT)])

    @pl.when(s == NS - 1)
    def _():
        pltpu.sync_copy(agg_sh.at[pl.ds(s * RT, RTL)],
                        agg_out.at[c, pl.ds(s * RT, RTL)])
    if with_cnt:
        @pl.when(s < N // CT)
        def _():
            pltpu.sync_copy(cnt_sh.at[pl.ds(s * CT, CT)], zv)
            pltpu.sync_copy(zv, cnt_out.at[pl.ds(c * N + s * CT, CT)])


def _make_sc_agg(with_cnt):
    mesh = plsc.VectorSubcoreMesh(core_axis_name="c", subcore_axis_name="s",
                                  num_cores=NC, num_subcores=NS)
    return pl.kernel(
        functools.partial(_sc_agg_body, with_cnt),
        out_type=(
            jax.ShapeDtypeStruct((NC, N, F), jnp.float32),
            jax.ShapeDtypeStruct((NC * N,), jnp.float32),
        ),
        mesh=mesh,
        scratch_types=[
            pltpu.VMEM_SHARED((N, F), jnp.float32),
            pltpu.VMEM_SHARED((N,), jnp.float32),
            pltpu.VMEM((ET,), jnp.int32),
            pltpu.VMEM((CH,), jnp.int32),
            pltpu.VMEM((CH,), jnp.int32),
            pltpu.VMEM((CH, F), jnp.float32),
            pltpu.VMEM((CH, F), jnp.float32),
            pltpu.VMEM((ONES,), jnp.float32),
            pltpu.VMEM((CT,), jnp.float32),
            pltpu.SemaphoreType.DMA,
            pltpu.SemaphoreType.DMA,
            pltpu.SemaphoreType.DMA,
            pltpu.SemaphoreType.DMA,
            pltpu.SemaphoreType.DMA,
            pltpu.SemaphoreType.DMA,
            pltpu.SemaphoreType.DMA,
            pltpu.SemaphoreType.DMA,
        ],
        name="sage_sc_agg" + ("_cnt" if with_cnt else ""),
    )


_sc_agg_cnt = _make_sc_agg(True)
_sc_agg = _make_sc_agg(False)

BR = 2000


def _tc_self_body(x_ref, w_ref, o_ref):
    o_ref[...] = jnp.dot(x_ref[...], w_ref[...],
                         preferred_element_type=jnp.float32)


def _tc1_body(agg_ref, cnt_ref, self_ref, wl_ref, bl_ref,
              gm_ref, bt_ref, rm_ref, rv_ref, o_ref):
    agg = agg_ref[0] + agg_ref[1]
    cnt = cnt_ref[:, 0:1] + cnt_ref[:, 1:2]
    rinv = 1.0 / jnp.maximum(cnt, 1.0)
    z = (jnp.dot(agg * rinv, wl_ref[...],
                 preferred_element_type=jnp.float32)
         + self_ref[...] + bl_ref[...])
    sc = gm_ref[...] * lax.rsqrt(rv_ref[...] + 1e-5)
    sh = bt_ref[...] - rm_ref[...] * sc
    o_ref[...] = jnp.maximum(z * sc + sh, 0.0)


def _tc2_body(agg_ref, cnt_ref, self_ref, wl_ref, bl_ref, o_ref):
    agg = agg_ref[0] + agg_ref[1]
    cnt = cnt_ref[:, 0:1] + cnt_ref[:, 1:2]
    rinv = 1.0 / jnp.maximum(cnt, 1.0)
    o_ref[...] = (jnp.dot(agg * rinv, wl_ref[...],
                          preferred_element_type=jnp.float32)
                  + self_ref[...] + bl_ref[...])


_row_spec = pl.BlockSpec((BR, F), lambda i: (i, 0))
_agg_spec = pl.BlockSpec((NC, BR, F), lambda i: (0, i, 0))
_cnt_spec = pl.BlockSpec((BR, NC), lambda i: (i, 0))
_mat_spec = pl.BlockSpec((F, F), lambda i: (0, 0))
_vec_spec = pl.BlockSpec((1, F), lambda i: (0, 0))


def _tc_self(x, w):
    return pl.pallas_call(
        _tc_self_body,
        grid=(N // BR,),
        in_specs=[_row_spec, _mat_spec],
        out_specs=_row_spec,
        out_shape=jax.ShapeDtypeStruct((N, F), jnp.float32),
    )(x, w)


def _tc1(agg, cnt, self1, wl, bl, gm, bt, rm, rv):
    return pl.pallas_call(
        _tc1_body,
        grid=(N // BR,),
        in_specs=[_agg_spec, _cnt_spec, _row_spec, _mat_spec, _vec_spec,
                  _vec_spec, _vec_spec, _vec_spec, _vec_spec],
        out_specs=_row_spec,
        out_shape=jax.ShapeDtypeStruct((N, F), jnp.float32),
    )(agg, cnt, self1, wl, bl, gm, bt, rm, rv)


def _tc2(agg, cnt, self2, wl, bl):
    return pl.pallas_call(
        _tc2_body,
        grid=(N // BR,),
        in_specs=[_agg_spec, _cnt_spec, _row_spec, _mat_spec, _vec_spec],
        out_specs=_row_spec,
        out_shape=jax.ShapeDtypeStruct((N, F), jnp.float32),
    )(agg, cnt, self2, wl, bl)


def kernel(x, ei, W1l, b1l, W1r, gamma, beta, rm, rv, W2l, b2l, W2r):
    eidx = ei.reshape(2 * E)

    agg1, cnt = _sc_agg_cnt(x, eidx)
    self1 = _tc_self(x, W1r)
    cnt2 = cnt.reshape(NC, N).T
    h = _tc1(agg1, cnt2, self1, W1l, b1l.reshape(1, F),
             gamma.reshape(1, F), beta.reshape(1, F),
             rm.reshape(1, F), rv.reshape(1, F))

    agg2, _ = _sc_agg(h, eidx)
    self2 = _tc_self(h, W2r)
    out = _tc2(agg2, cnt2, self2, W2l, b2l.reshape(1, F))
    return out
